# baseline (speedup 1.0000x reference)
import functools

import jax
import jax.numpy as jnp
from jax import lax
from jax.experimental import pallas as pl
from jax.experimental.pallas import tpu as pltpu
from jax.experimental.pallas import tpu_sc as plsc

_VOCAB = 100000
_HIDDEN = 64
_BATCH = 1024

_VBLK = 4096


@functools.lru_cache(maxsize=None)
def _sc_gather_fn():
    info = plsc.get_sparse_core_info()
    nc, ns, nl = info.num_cores, info.num_subcores, info.num_lanes
    nw = nc * ns
    d_per_w = _HIDDEN // nw
    mesh = plsc.VectorSubcoreMesh(core_axis_name="c", subcore_axis_name="s")

    @functools.partial(
        pl.kernel,
        mesh=mesh,
        out_type=jax.ShapeDtypeStruct((_HIDDEN, _BATCH), jnp.float32),
        compiler_params=pltpu.CompilerParams(needs_layout_passes=False),
        scratch_types=[
            pltpu.VMEM((_BATCH,), jnp.int32),
            pltpu.VMEM((_VOCAB,), jnp.float32),
            pltpu.VMEM((_BATCH,), jnp.float32),
            pltpu.SemaphoreType.DMA,
        ],
    )
    def gather(tableT_hbm, idx_hbm, outT_hbm, x_v, row_v, out_v, sem):
        wid = lax.axis_index("s") * nc + lax.axis_index("c")
        d0 = wid * d_per_w
        row_cp = pltpu.async_copy(tableT_hbm.at[d0], row_v, sem)
        pltpu.sync_copy(idx_hbm, x_v)
        for k in range(d_per_w):
            d = d0 + k
            row_cp.wait()
            for i in range(_BATCH // nl):
                sl = pl.ds(i * nl, nl)
                out_v[sl] = plsc.load_gather(row_v, [x_v[sl]])
            if k + 1 < d_per_w:
                row_cp = pltpu.async_copy(tableT_hbm.at[d + 1], row_v, sem)
            pltpu.sync_copy(out_v, outT_hbm.at[d])

    return gather


def _matmul_body(wt_ref, ht_ref, o_ref):
    o_ref[...] = lax.dot_general(
        wt_ref[...], ht_ref[...],
        dimension_numbers=(((0,), (0,)), ((), ())),
        preferred_element_type=jnp.float32,
    )


def kernel(x, embed_table, W_out):
    hT = _sc_gather_fn()(embed_table.T, x.astype(jnp.int32))
    grid = pl.cdiv(_VOCAB, _VBLK)
    logitsT = pl.pallas_call(
        _matmul_body,
        grid=(grid,),
        in_specs=[
            pl.BlockSpec((_HIDDEN, _VBLK), lambda j: (0, j)),
            pl.BlockSpec((_HIDDEN, _BATCH), lambda j: (0, 0)),
        ],
        out_specs=pl.BlockSpec((_VBLK, _BATCH), lambda j: (j, 0)),
        out_shape=jax.ShapeDtypeStruct((_VOCAB, _BATCH), jnp.float32),
        compiler_params=pltpu.CompilerParams(
            dimension_semantics=("parallel",),
        ),
    )(W_out.T, hT)
    return logitsT.T


# speedup vs baseline: 1.0022x; 1.0022x over previous
import functools

import jax
import jax.numpy as jnp
from jax import lax
from jax.experimental import pallas as pl
from jax.experimental.pallas import tpu as pltpu
from jax.experimental.pallas import tpu_sc as plsc

_VOCAB = 100000
_HIDDEN = 64
_BATCH = 1024

_VBLK = 4096


@functools.lru_cache(maxsize=None)
def _sc_gather_fn():
    info = plsc.get_sparse_core_info()
    nc, ns, nl = info.num_cores, info.num_subcores, info.num_lanes
    nw = nc * ns
    d_per_w = _HIDDEN // nw
    mesh = plsc.VectorSubcoreMesh(core_axis_name="c", subcore_axis_name="s")

    @functools.partial(
        pl.kernel,
        mesh=mesh,
        out_type=jax.ShapeDtypeStruct((_HIDDEN, _BATCH), jnp.float32),
        compiler_params=pltpu.CompilerParams(needs_layout_passes=False),
        scratch_types=[
            pltpu.VMEM((_BATCH,), jnp.int32),
            pltpu.VMEM((_VOCAB,), jnp.float32),
            pltpu.VMEM((d_per_w, _BATCH), jnp.float32),
            pltpu.SemaphoreType.DMA,
        ],
    )
    def gather(tableT_hbm, idx_hbm, outT_hbm, x_v, row_v, out_v, sem):
        wid = lax.axis_index("s") * nc + lax.axis_index("c")
        d0 = wid * d_per_w
        row_cp = pltpu.async_copy(tableT_hbm.at[d0], row_v, sem)
        pltpu.sync_copy(idx_hbm, x_v)
        for k in range(d_per_w):
            row_cp.wait()
            for i in range(_BATCH // nl):
                sl = pl.ds(i * nl, nl)
                out_v[k, sl] = plsc.load_gather(row_v, [x_v[sl]])
            if k + 1 < d_per_w:
                row_cp = pltpu.async_copy(tableT_hbm.at[d0 + k + 1], row_v, sem)
        pltpu.sync_copy(out_v, outT_hbm.at[pl.ds(d0, d_per_w)])

    return gather


def _matmul_body(wt_ref, ht_ref, o_ref):
    o_ref[...] = lax.dot_general(
        wt_ref[...], ht_ref[...],
        dimension_numbers=(((0,), (0,)), ((), ())),
        preferred_element_type=jnp.float32,
    )


def kernel(x, embed_table, W_out):
    hT = _sc_gather_fn()(embed_table.T, x.astype(jnp.int32))
    grid = pl.cdiv(_VOCAB, _VBLK)
    logitsT = pl.pallas_call(
        _matmul_body,
        grid=(grid,),
        in_specs=[
            pl.BlockSpec((_HIDDEN, _VBLK), lambda j: (0, j)),
            pl.BlockSpec((_HIDDEN, _BATCH), lambda j: (0, 0)),
        ],
        out_specs=pl.BlockSpec((_VBLK, _BATCH), lambda j: (j, 0)),
        out_shape=jax.ShapeDtypeStruct((_VOCAB, _BATCH), jnp.float32),
        compiler_params=pltpu.CompilerParams(
            dimension_semantics=("parallel",),
        ),
    )(W_out.T, hT)
    return logitsT.T
